# baseline (speedup 1.0000x reference)
import functools

import jax
import jax.numpy as jnp
from jax import lax
from jax.experimental import pallas as pl
from jax.experimental.pallas import tpu as pltpu
from jax.experimental.pallas import tpu_sc as plsc

TIME_DIM = 64
HOUR_SIZE = 24
DAY_SIZE = 32
MONTH_SIZE = 13
N = 16384

NUM_CORES = 2
NUM_SUBCORES = 16
LANES = 16
NW = NUM_CORES * NUM_SUBCORES
B_PER_W = N // NW
CHUNK = 128
N_CHUNKS = B_PER_W // CHUNK

_FIELDS = ((2, 0, HOUR_SIZE), (1, HOUR_SIZE, DAY_SIZE),
           (0, HOUR_SIZE + DAY_SIZE, MONTH_SIZE))


def _body(x_hbm, tab_hbm, out_hbm, x_v, idx_v, b0, b1, b2, sem):
    wid = lax.axis_index("s") * NUM_CORES + lax.axis_index("c")
    base = wid * B_PER_W

    for c, (col, _off, _size) in enumerate(_FIELDS):
        pltpu.sync_copy(x_hbm.at[pl.ds(col * N + base, B_PER_W)],
                        x_v.at[pl.ds(c * B_PER_W, B_PER_W)])

    for c, (col, off, size) in enumerate(_FIELDS):
        for k in range(N_CHUNKS):
            for g in range(CHUNK // LANES):
                row0 = k * CHUNK + g * LANES
                vals = x_v[pl.ds(c * B_PER_W + row0, LANES)]
                idx = ((vals + 0.5) * float(size)).astype(jnp.int32)
                idx = jnp.clip(idx, 0, size - 1) + off
                idx_v[pl.ds((c * N_CHUNKS + k) * CHUNK + g * LANES, LANES)] = idx

    bufs = (b0, b1, b2)
    descs = []
    for c in range(3):
        for k in range(N_CHUNKS):
            descs.append(pltpu.async_copy(
                tab_hbm.at[idx_v.at[pl.ds((c * N_CHUNKS + k) * CHUNK, CHUNK)]],
                bufs[c].at[pl.ds(k * CHUNK, CHUNK)], sem))
    for d in descs:
        d.wait()

    def add_row(r, carry):
        for j in range(TIME_DIM // LANES):
            s = pl.ds(j * LANES, LANES)
            b0[r, s] = b0[r, s] + b1[r, s] + b2[r, s]
        return carry

    lax.fori_loop(0, B_PER_W, add_row, 0)

    pltpu.sync_copy(b0, out_hbm.at[pl.ds(base, B_PER_W)])


@jax.jit
def _lookup(x_flat, tab):
    mesh = plsc.VectorSubcoreMesh(core_axis_name="c", subcore_axis_name="s")
    run = pl.kernel(
        _body,
        out_type=jax.ShapeDtypeStruct((N, TIME_DIM), jnp.float32),
        mesh=mesh,
        scratch_types=[
            pltpu.VMEM((3 * B_PER_W,), jnp.float32),
            pltpu.VMEM((3 * B_PER_W,), jnp.int32),
            pltpu.VMEM((B_PER_W, TIME_DIM), jnp.float32),
            pltpu.VMEM((B_PER_W, TIME_DIM), jnp.float32),
            pltpu.VMEM((B_PER_W, TIME_DIM), jnp.float32),
            pltpu.SemaphoreType.DMA,
        ],
        compiler_params=pltpu.CompilerParams(use_tc_tiling_on_sc=False),
    )
    return run(x_flat, tab)


def kernel(x, hour_embed, day_embed, month_embed):
    tab = jnp.concatenate([hour_embed, day_embed, month_embed], axis=0)
    return _lookup(x.T.reshape(-1), tab)


# speedup vs baseline: 15.1120x; 15.1120x over previous
import jax
import jax.numpy as jnp
from jax import lax
from jax.experimental import pallas as pl
from jax.experimental.pallas import tpu as pltpu
from jax.experimental.pallas import tpu_sc as plsc

TIME_DIM = 64
HOUR_SIZE = 24
DAY_SIZE = 32
MONTH_SIZE = 13
N = 16384
TAB_ROWS = HOUR_SIZE + DAY_SIZE + MONTH_SIZE

NUM_CORES = 2
NUM_SUBCORES = 16
LANES = 16
NW = NUM_CORES * NUM_SUBCORES
B_PER_W = N // NW

_FIELDS = ((2, 0, HOUR_SIZE), (1, HOUR_SIZE, DAY_SIZE),
           (0, HOUR_SIZE + DAY_SIZE, MONTH_SIZE))


def _body(x_hbm, tab_hbm, out_hbm, tab_v, x_v, idx_v, out_v, sem):
    wid = lax.axis_index("s") * NUM_CORES + lax.axis_index("c")
    base = wid * B_PER_W

    tab_cp = pltpu.async_copy(tab_hbm, tab_v, sem)
    x_cps = [
        pltpu.async_copy(x_hbm.at[pl.ds(col * N + base, B_PER_W)],
                         x_v.at[pl.ds(c * B_PER_W, B_PER_W)], sem)
        for c, (col, _off, _size) in enumerate(_FIELDS)
    ]
    tab_cp.wait()
    for cp in x_cps:
        cp.wait()

    for c, (col, off, size) in enumerate(_FIELDS):
        for g in range(B_PER_W // LANES):
            vals = x_v[pl.ds(c * B_PER_W + g * LANES, LANES)]
            idx = ((vals + 0.5) * float(size)).astype(jnp.int32)
            idx = (jnp.clip(idx, 0, size - 1) + off) * TIME_DIM
            idx_v[pl.ds(c * B_PER_W + g * LANES, LANES)] = idx

    def group(g, carry):
        iv0 = idx_v[pl.ds(g * LANES, LANES)]
        iv1 = idx_v[pl.ds(B_PER_W + g * LANES, LANES)]
        iv2 = idx_v[pl.ds(2 * B_PER_W + g * LANES, LANES)]
        for l in range(LANES):
            r = g * LANES + l
            i0, i1, i2 = iv0[l], iv1[l], iv2[l]
            for j in range(TIME_DIM // LANES):
                o = j * LANES
                out_v[r, pl.ds(o, LANES)] = (
                    tab_v[pl.ds(i0 + o, LANES)]
                    + tab_v[pl.ds(i1 + o, LANES)]
                    + tab_v[pl.ds(i2 + o, LANES)])
        return carry

    lax.fori_loop(0, B_PER_W // LANES, group, 0)

    pltpu.sync_copy(out_v, out_hbm.at[pl.ds(base, B_PER_W)])


@jax.jit
def _lookup(x_flat, tab_flat):
    mesh = plsc.VectorSubcoreMesh(core_axis_name="c", subcore_axis_name="s")
    run = pl.kernel(
        _body,
        out_type=jax.ShapeDtypeStruct((N, TIME_DIM), jnp.float32),
        mesh=mesh,
        scratch_types=[
            pltpu.VMEM((TAB_ROWS * TIME_DIM,), jnp.float32),
            pltpu.VMEM((3 * B_PER_W,), jnp.float32),
            pltpu.VMEM((3 * B_PER_W,), jnp.int32),
            pltpu.VMEM((B_PER_W, TIME_DIM), jnp.float32),
            pltpu.SemaphoreType.DMA,
        ],
        compiler_params=pltpu.CompilerParams(use_tc_tiling_on_sc=False),
    )
    return run(x_flat, tab_flat)


def kernel(x, hour_embed, day_embed, month_embed):
    tab = jnp.concatenate([hour_embed, day_embed, month_embed], axis=0)
    return _lookup(x.T.reshape(-1), tab.reshape(-1))


# speedup vs baseline: 19.2132x; 1.2714x over previous
import jax
import jax.numpy as jnp
from jax import lax
from jax.experimental import pallas as pl
from jax.experimental.pallas import tpu as pltpu
from jax.experimental.pallas import tpu_sc as plsc

TIME_DIM = 64
HOUR_SIZE = 24
DAY_SIZE = 32
MONTH_SIZE = 13
N = 16384
TAB_ROWS = HOUR_SIZE + DAY_SIZE + MONTH_SIZE

NUM_CORES = 2
NUM_SUBCORES = 16
LANES = 16
NW = NUM_CORES * NUM_SUBCORES
B_PER_W = N // NW
CHUNK = 128

_FIELDS = ((2, 0, HOUR_SIZE), (1, HOUR_SIZE, DAY_SIZE),
           (0, HOUR_SIZE + DAY_SIZE, MONTH_SIZE))


def _body(x_hbm, tab_hbm, out_hbm, tab_v, x_v, idx_v, out_v, sem):
    wid = lax.axis_index("s") * NUM_CORES + lax.axis_index("c")
    base = wid * B_PER_W

    tab_cp = pltpu.async_copy(tab_hbm, tab_v, sem)
    x_cps = [
        pltpu.async_copy(x_hbm.at[pl.ds(col * N + base, B_PER_W)],
                         x_v.at[pl.ds(c * B_PER_W, B_PER_W)], sem)
        for c, (col, _off, _size) in enumerate(_FIELDS)
    ]
    tab_cp.wait()
    for cp in x_cps:
        cp.wait()

    for c, (col, off, size) in enumerate(_FIELDS):
        for g in range(B_PER_W // LANES):
            vals = x_v[pl.ds(c * B_PER_W + g * LANES, LANES)]
            idx = ((vals + 0.5) * float(size)).astype(jnp.int32)
            idx = (jnp.clip(idx, 0, size - 1) + off) * TIME_DIM
            idx_v[pl.ds(c * B_PER_W + g * LANES, LANES)] = idx

    def group(g, carry):
        iv0 = idx_v[pl.ds(g * LANES, LANES)]
        iv1 = idx_v[pl.ds(B_PER_W + g * LANES, LANES)]
        iv2 = idx_v[pl.ds(2 * B_PER_W + g * LANES, LANES)]
        i0, i1, i2 = iv0[0], iv1[0], iv2[0]
        eq = (plsc.all_reduce_population_count(iv0 == i0)
              + plsc.all_reduce_population_count(iv1 == i1)
              + plsc.all_reduce_population_count(iv2 == i2))
        uniform = eq[0] == 3 * LANES

        @pl.when(uniform)
        def _fast():
            rows = [tab_v[pl.ds(i0 + j * LANES, LANES)]
                    + tab_v[pl.ds(i1 + j * LANES, LANES)]
                    + tab_v[pl.ds(i2 + j * LANES, LANES)]
                    for j in range(TIME_DIM // LANES)]
            for l in range(LANES):
                for j in range(TIME_DIM // LANES):
                    out_v[g * LANES + l, pl.ds(j * LANES, LANES)] = rows[j]

        @pl.when(jnp.logical_not(uniform))
        def _slow():
            for l in range(LANES):
                r = g * LANES + l
                a0, a1, a2 = iv0[l], iv1[l], iv2[l]
                for j in range(TIME_DIM // LANES):
                    o = j * LANES
                    out_v[r, pl.ds(o, LANES)] = (
                        tab_v[pl.ds(a0 + o, LANES)]
                        + tab_v[pl.ds(a1 + o, LANES)]
                        + tab_v[pl.ds(a2 + o, LANES)])
        return carry

    lax.fori_loop(0, B_PER_W // LANES, group, 0)

    pltpu.sync_copy(out_v, out_hbm.at[pl.ds(base, B_PER_W)])


@jax.jit
def _lookup(x_flat, tab_flat):
    mesh = plsc.VectorSubcoreMesh(core_axis_name="c", subcore_axis_name="s")
    run = pl.kernel(
        _body,
        out_type=jax.ShapeDtypeStruct((N, TIME_DIM), jnp.float32),
        mesh=mesh,
        scratch_types=[
            pltpu.VMEM((TAB_ROWS * TIME_DIM,), jnp.float32),
            pltpu.VMEM((3 * B_PER_W,), jnp.float32),
            pltpu.VMEM((3 * B_PER_W,), jnp.int32),
            pltpu.VMEM((B_PER_W, TIME_DIM), jnp.float32),
            pltpu.SemaphoreType.DMA,
        ],
        compiler_params=pltpu.CompilerParams(
            use_tc_tiling_on_sc=False, needs_layout_passes=False),
    )
    return run(x_flat, tab_flat)


def kernel(x, hour_embed, day_embed, month_embed):
    tab = jnp.concatenate([hour_embed, day_embed, month_embed], axis=0)
    return _lookup(x.T.reshape(-1), tab.reshape(-1))
